# Initial kernel scaffold; baseline (speedup 1.0000x reference)
import jax
import jax.numpy as jnp
from jax.experimental import pallas as pl


def kernel(inputs, hidden_state, edge_row, edge_col, lap_val, W_gate, b_gate, W_cand, b_cand, W_proj, b_proj):
    raise NotImplementedError("write your pallas kernel here")


# speedup vs baseline: 1.1540x; 1.1540x over previous
import functools

import jax
import jax.numpy as jnp
from jax import lax
from jax.experimental import pallas as pl
from jax.experimental.pallas import tpu as pltpu
from jax.experimental.pallas import tpu_sc as plsc

N = 10000
E = 320000
B = 8
U = 64
F = 128
GATE = 2 * U

BN = 2000

NC = 2
NS = 16
SROWS = 624
TAIL = N - NS * SROWS
SB = 48
EPT = E // NS
EW = 80
NWIN = EPT // EW
CPS = B // NC


def _gate_body(x0_ref, x1_ref, x2_ref, hx_ref, w_ref, b_ref, xc_ref, u_ref):
    acc = (
        jnp.dot(x0_ref[...], w_ref[0], preferred_element_type=jnp.float32)
        + jnp.dot(x1_ref[...], w_ref[1], preferred_element_type=jnp.float32)
        + jnp.dot(x2_ref[...], w_ref[2], preferred_element_type=jnp.float32)
        + b_ref[...]
    )
    v = jax.nn.sigmoid(acc)
    r = v[:, :U]
    u = v[:, U:]
    u_ref[...] = u
    rh = r * hx_ref[...]
    xc_ref[...] = jnp.concatenate(
        [x0_ref[:, 0:1], rh, jnp.zeros((BN, F - 1 - U), jnp.float32)], axis=1
    )


def _cand_body(x0_ref, x1_ref, x2_ref, u_ref, hx_ref, w_ref, b_ref, wp_ref,
               bp_ref, h_ref, p_ref):
    acc = (
        jnp.dot(x0_ref[...], w_ref[0], preferred_element_type=jnp.float32)
        + jnp.dot(x1_ref[...], w_ref[1], preferred_element_type=jnp.float32)
        + jnp.dot(x2_ref[...], w_ref[2], preferred_element_type=jnp.float32)
        + b_ref[...]
    )
    c = jnp.tanh(acc)
    u = u_ref[...]
    h = u * hx_ref[...] + (1.0 - u) * c
    h_ref[...] = h
    p_ref[...] = jnp.dot(h, wp_ref[...], preferred_element_type=jnp.float32) + bp_ref[...]


def _row_spec(width):
    return pl.BlockSpec((BN, width), lambda i: (i, 0))


def _full_spec(shape):
    return pl.BlockSpec(shape, lambda i: tuple(0 for _ in shape))


def _gate_call(x0, x1, x2, hx, w, b):
    grid = (8 * N // BN,)
    return pl.pallas_call(
        _gate_body,
        grid=grid,
        in_specs=[
            _row_spec(F), _row_spec(F), _row_spec(F), _row_spec(U),
            _full_spec((3, F, GATE)), _full_spec((1, GATE)),
        ],
        out_specs=[_row_spec(F), _row_spec(U)],
        out_shape=[
            jax.ShapeDtypeStruct((8 * N, F), jnp.float32),
            jax.ShapeDtypeStruct((8 * N, U), jnp.float32),
        ],
    )(x0, x1, x2, hx, w, b)


def _cand_call(x0, x1, x2, u, hx, w, b, wp, bp):
    grid = (8 * N // BN,)
    return pl.pallas_call(
        _cand_body,
        grid=grid,
        in_specs=[
            _row_spec(F), _row_spec(F), _row_spec(F), _row_spec(U), _row_spec(U),
            _full_spec((3, F, U)), _full_spec((1, U)),
            _full_spec((U, 1)), _full_spec((1, 1)),
        ],
        out_specs=[_row_spec(U), _row_spec(1)],
        out_shape=[
            jax.ShapeDtypeStruct((8 * N, U), jnp.float32),
            jax.ShapeDtypeStruct((8 * N, 1), jnp.float32),
        ],
    )(x0, x1, x2, u, hx, w, b, wp, bp)


def _spmm_sc_body(x_hbm, row_hbm, col_hbm, val_hbm, y1_hbm, y2_hbm,
                  acc, sbuf, colb, colgb, rowb, rowsb, valb, gsem):
    ci = lax.axis_index("c")
    si = lax.axis_index("s")
    t0 = si * SROWS
    e0 = si * EPT
    nf = F // 16

    def edge_pass(src_hbm, acc, base, scale):
        def win(w, carry):
            eb = e0 + w * EW
            pltpu.sync_copy(col_hbm.at[pl.ds(eb, EW)], colb)
            pltpu.sync_copy(row_hbm.at[pl.ds(eb, EW)], rowb)
            pltpu.sync_copy(val_hbm.at[pl.ds(eb, EW)], valb)
            for jj in range(EW // 16):
                sl = pl.ds(jj * 16, 16)
                colgb[sl] = colb[sl] + base
            pltpu.async_copy(src_hbm.at[colgb], rowsb, gsem).wait()

            def scl(k, c2):
                kk = jnp.full((16,), k, jnp.int32)
                v = plsc.load_gather(valb, [kk]) * scale
                for jj in range(nf):
                    sl = pl.ds(jj * 16, 16)
                    rowsb[k, sl] = rowsb[k, sl] * v
                return c2

            lax.fori_loop(0, EW, scl, 0)
            pltpu.sync_copy(rowsb, acc.at[rowb], add=True)
            return carry

        lax.fori_loop(0, NWIN, win, 0)

    def zero_stripe(off, size):
        def zero_row(rr, c2):
            for jj in range(nf):
                sbuf[rr, pl.ds(jj * 16, 16)] = jnp.zeros((16,), jnp.float32)
            return c2

        lax.fori_loop(0, min(size, SB), zero_row, 0)

        def blk(i, c2):
            pltpu.sync_copy(sbuf.at[pl.ds(0, min(size, SB))],
                            acc.at[pl.ds(off + i * SB, min(size, SB))])
            return c2

        lax.fori_loop(0, pl.cdiv(size, SB), blk, 0)

    def negx_stripe(base, off, size):
        bs = min(size, SB)

        def blk(i, c2):
            pltpu.sync_copy(x_hbm.at[pl.ds(base + off + i * SB, bs)],
                            sbuf.at[pl.ds(0, bs)])

            def neg_row(rr, c3):
                for jj in range(nf):
                    sl = pl.ds(jj * 16, 16)
                    sbuf[rr, sl] = -sbuf[rr, sl]
                return c3

            lax.fori_loop(0, bs, neg_row, 0)
            pltpu.sync_copy(sbuf.at[pl.ds(0, bs)], acc.at[pl.ds(off + i * SB, bs)])
            return c2

        lax.fori_loop(0, pl.cdiv(size, SB), blk, 0)

    def writeback_stripe(y_hbm, base, off, size):
        pltpu.sync_copy(acc.at[pl.ds(off, size)], y_hbm.at[pl.ds(base + off, size)])

    def all_stripes(fn, *args):
        fn(*args, t0, SROWS)

        @pl.when(si == 0)
        def _():
            fn(*args, NS * SROWS, TAIL)

    def chunk(j, carry):
        base = (ci * CPS + j) * N

        all_stripes(zero_stripe)
        plsc.subcore_barrier()
        edge_pass(x_hbm, acc, base, 1.0)
        plsc.subcore_barrier()
        all_stripes(writeback_stripe, y1_hbm, base)
        plsc.subcore_barrier()

        all_stripes(negx_stripe, base)
        plsc.subcore_barrier()
        edge_pass(y1_hbm, acc, base, 2.0)
        plsc.subcore_barrier()
        all_stripes(writeback_stripe, y2_hbm, base)
        plsc.subcore_barrier()
        return carry

    lax.fori_loop(0, CPS, chunk, 0)


_spmm_sc_call = pl.kernel(
    _spmm_sc_body,
    out_type=[
        jax.ShapeDtypeStruct((B * N, F), jnp.float32),
        jax.ShapeDtypeStruct((B * N, F), jnp.float32),
    ],
    mesh=plsc.VectorSubcoreMesh(core_axis_name="c", subcore_axis_name="s"),
    compiler_params=pltpu.CompilerParams(needs_layout_passes=False),
    scratch_types=[
        pltpu.VMEM_SHARED((N, F), jnp.float32),
        pltpu.VMEM((SB, F), jnp.float32),
        pltpu.VMEM((EW,), jnp.int32),
        pltpu.VMEM((EW,), jnp.int32),
        pltpu.VMEM((EW,), jnp.int32),
        pltpu.VMEM((EW, F), jnp.float32),
        pltpu.VMEM((EW,), jnp.float32),
        pltpu.SemaphoreType.DMA,
    ],
)


def _spmm_pair(row, col, val, x):
    y1, y2 = _spmm_sc_call(x.reshape(B * N, F), row, col, val)
    return y1.reshape(B, N, F), y2.reshape(B, N, F)


def _prep_w(W, out_dim):
    Wk = W.reshape(U + 1, 3, out_dim).transpose(1, 0, 2)
    return jnp.pad(Wk, ((0, 0), (0, F - 1 - U), (0, 0)))


def kernel(inputs, hidden_state, edge_row, edge_col, lap_val, W_gate, b_gate,
           W_cand, b_cand, W_proj, b_proj):
    inp = inputs.reshape(B, N, 1)
    hx = hidden_state[0].reshape(B, N, U)
    x0 = jnp.concatenate([inp, hx, jnp.zeros((B, N, F - 1 - U), jnp.float32)], axis=2)

    wg = _prep_w(W_gate, GATE)
    wc = _prep_w(W_cand, U)

    x1, x2 = _spmm_pair(edge_row, edge_col, lap_val, x0)
    xc, u = _gate_call(
        x0.reshape(8 * N, F), x1.reshape(8 * N, F), x2.reshape(8 * N, F),
        hx.reshape(8 * N, U), wg, b_gate.reshape(1, GATE))

    xcb = xc.reshape(B, N, F)
    xc1, xc2 = _spmm_pair(edge_row, edge_col, lap_val, xcb)
    h, p = _cand_call(
        xc, xc1.reshape(8 * N, F), xc2.reshape(8 * N, F), u,
        hx.reshape(8 * N, U), wc, b_cand.reshape(1, U),
        W_proj, b_proj.reshape(1, 1))

    out = p.reshape(B, N)
    hidden = h.reshape(1, B, N * U)
    return (out, hidden)


# speedup vs baseline: 1.9549x; 1.6940x over previous
import functools

import jax
import jax.numpy as jnp
from jax import lax
from jax.experimental import pallas as pl
from jax.experimental.pallas import tpu as pltpu
from jax.experimental.pallas import tpu_sc as plsc

N = 10000
E = 320000
B = 8
U = 64
F = 128
GATE = 2 * U

BN = 2000

NC = 2
NS = 16
SROWS = 624
TAIL = N - NS * SROWS
SB = 48
EPT = E // NS
EW = 40
NWIN = EPT // EW
NSLOT = 4
NFS = 5
CPS = B // NC


def _gate_body(x0_ref, x1_ref, x2_ref, hx_ref, w_ref, b_ref, xc_ref, u_ref):
    acc = (
        jnp.dot(x0_ref[...], w_ref[0], preferred_element_type=jnp.float32)
        + jnp.dot(x1_ref[...], w_ref[1], preferred_element_type=jnp.float32)
        + jnp.dot(x2_ref[...], w_ref[2], preferred_element_type=jnp.float32)
        + b_ref[...]
    )
    v = jax.nn.sigmoid(acc)
    r = v[:, :U]
    u = v[:, U:]
    u_ref[...] = u
    rh = r * hx_ref[...]
    xc_ref[...] = jnp.concatenate(
        [x0_ref[:, 0:1], rh, jnp.zeros((BN, F - 1 - U), jnp.float32)], axis=1
    )


def _cand_body(x0_ref, x1_ref, x2_ref, u_ref, hx_ref, w_ref, b_ref, wp_ref,
               bp_ref, h_ref, p_ref):
    acc = (
        jnp.dot(x0_ref[...], w_ref[0], preferred_element_type=jnp.float32)
        + jnp.dot(x1_ref[...], w_ref[1], preferred_element_type=jnp.float32)
        + jnp.dot(x2_ref[...], w_ref[2], preferred_element_type=jnp.float32)
        + b_ref[...]
    )
    c = jnp.tanh(acc)
    u = u_ref[...]
    h = u * hx_ref[...] + (1.0 - u) * c
    h_ref[...] = h
    p_ref[...] = jnp.dot(h, wp_ref[...], preferred_element_type=jnp.float32) + bp_ref[...]


def _row_spec(width):
    return pl.BlockSpec((BN, width), lambda i: (i, 0))


def _full_spec(shape):
    return pl.BlockSpec(shape, lambda i: tuple(0 for _ in shape))


def _gate_call(x0, x1, x2, hx, w, b):
    grid = (8 * N // BN,)
    return pl.pallas_call(
        _gate_body,
        grid=grid,
        in_specs=[
            _row_spec(F), _row_spec(F), _row_spec(F), _row_spec(U),
            _full_spec((3, F, GATE)), _full_spec((1, GATE)),
        ],
        out_specs=[_row_spec(F), _row_spec(U)],
        out_shape=[
            jax.ShapeDtypeStruct((8 * N, F), jnp.float32),
            jax.ShapeDtypeStruct((8 * N, U), jnp.float32),
        ],
    )(x0, x1, x2, hx, w, b)


def _cand_call(x0, x1, x2, u, hx, w, b, wp, bp):
    grid = (8 * N // BN,)
    return pl.pallas_call(
        _cand_body,
        grid=grid,
        in_specs=[
            _row_spec(F), _row_spec(F), _row_spec(F), _row_spec(U), _row_spec(U),
            _full_spec((3, F, U)), _full_spec((1, U)),
            _full_spec((U, 1)), _full_spec((1, 1)),
        ],
        out_specs=[_row_spec(U), _row_spec(1)],
        out_shape=[
            jax.ShapeDtypeStruct((8 * N, U), jnp.float32),
            jax.ShapeDtypeStruct((8 * N, 1), jnp.float32),
        ],
    )(x0, x1, x2, u, hx, w, b, wp, bp)


def _spmm_sc_body(x_hbm, colg_hbm, row_hbm, val_hbm, val2_hbm, y1_hbm, y2_hbm,
                  acc, sbuf, colbs, rowbs, valbs, rowsbs, sclbs,
                  isems, gsems, ssems):
    ci = lax.axis_index("c")
    si = lax.axis_index("s")
    t0 = si * SROWS
    e0 = si * EPT
    nf = F // 16

    for s_init in range(NSLOT):
        def pad_row(rr, c2, _sb=sclbs[s_init]):
            for jj in range(NFS, nf):
                _sb[rr, pl.ds(jj * 16, 16)] = jnp.zeros((16,), jnp.float32)
            return c2

        lax.fori_loop(0, EW, pad_row, 0)

    def edge_pass(src_hbm, bidx, vhbm):
        cbase = bidx * E + e0

        def in_args(w, s):
            return (
                (colg_hbm.at[pl.ds(cbase + w * EW, EW)], colbs[s], isems[s]),
                (row_hbm.at[pl.ds(e0 + w * EW, EW)], rowbs[s], isems[s]),
                (vhbm.at[pl.ds(e0 + w * EW, EW)], valbs[s], isems[s]),
            )

        def issue_inputs(w, s):
            for a in in_args(w, s):
                pltpu.async_copy(*a)

        def wait_inputs(w, s):
            for a in in_args(w, s):
                pltpu.make_async_copy(*a).wait()

        def gather_start(s):
            pltpu.async_copy(src_hbm.at[colbs[s]], rowsbs[s], gsems[s])

        def gather_wait(s):
            pltpu.make_async_copy(src_hbm.at[colbs[s]], rowsbs[s], gsems[s]).wait()

        def scatter_start(s):
            pltpu.async_copy(sclbs[s], acc.at[rowbs[s]], ssems[s], add=True)

        def scatter_wait(s):
            pltpu.make_async_copy(sclbs[s], acc.at[rowbs[s]], ssems[s]).wait()

        def scale(s):
            rb, sb_ = rowsbs[s], sclbs[s]

            def scl(k, c2):
                kk = jnp.full((16,), k, jnp.int32)
                v = plsc.load_gather(valbs[s], [kk])
                for jj in range(NFS):
                    sl = pl.ds(jj * 16, 16)
                    sb_[k, sl] = rb[k, sl] * v
                return c2

            lax.fori_loop(0, EW, scl, 0)

        issue_inputs(0, 0)
        issue_inputs(1, 1)
        wait_inputs(0, 0)
        gather_start(0)

        def quad(q, carry):
            w0 = q * NSLOT
            for s_ in range(NSLOT):
                w = w0 + s_
                sp1 = (s_ + 1) % NSLOT
                sp2 = (s_ + 2) % NSLOT

                @pl.when(w >= 2)
                def _():
                    scatter_wait(sp2)

                @pl.when(w + 2 < NWIN)
                def _():
                    issue_inputs(w + 2, sp2)

                @pl.when(w + 1 < NWIN)
                def _():
                    wait_inputs(w + 1, sp1)
                    gather_start(sp1)

                gather_wait(s_)
                scale(s_)
                scatter_start(s_)
            return carry

        lax.fori_loop(0, NWIN // NSLOT, quad, 0)
        scatter_wait((NWIN - 2) % NSLOT)
        scatter_wait((NWIN - 1) % NSLOT)

    def zero_stripe(off, size):
        def zero_row(rr, c2):
            for jj in range(nf):
                sbuf[rr, pl.ds(jj * 16, 16)] = jnp.zeros((16,), jnp.float32)
            return c2

        lax.fori_loop(0, min(size, SB), zero_row, 0)

        def blk(i, c2):
            pltpu.sync_copy(sbuf.at[pl.ds(0, min(size, SB))],
                            acc.at[pl.ds(off + i * SB, min(size, SB))])
            return c2

        lax.fori_loop(0, pl.cdiv(size, SB), blk, 0)

    def negx_stripe(base, off, size):
        bs = min(size, SB)

        def blk(i, c2):
            pltpu.sync_copy(x_hbm.at[pl.ds(base + off + i * SB, bs)],
                            sbuf.at[pl.ds(0, bs)])

            def neg_row(rr, c3):
                for jj in range(nf):
                    sl = pl.ds(jj * 16, 16)
                    sbuf[rr, sl] = -sbuf[rr, sl]
                return c3

            lax.fori_loop(0, bs, neg_row, 0)
            pltpu.sync_copy(sbuf.at[pl.ds(0, bs)], acc.at[pl.ds(off + i * SB, bs)])
            return c2

        lax.fori_loop(0, pl.cdiv(size, SB), blk, 0)

    def writeback_stripe(y_hbm, base, off, size):
        pltpu.sync_copy(acc.at[pl.ds(off, size)], y_hbm.at[pl.ds(base + off, size)])

    def all_stripes(fn, *args):
        fn(*args, t0, SROWS)

        @pl.when(si == 0)
        def _():
            fn(*args, NS * SROWS, TAIL)

    def chunk(j, carry):
        bidx = ci * CPS + j
        base = bidx * N

        all_stripes(zero_stripe)
        plsc.subcore_barrier()
        edge_pass(x_hbm, bidx, val_hbm)
        plsc.subcore_barrier()
        all_stripes(writeback_stripe, y1_hbm, base)
        plsc.subcore_barrier()

        all_stripes(negx_stripe, base)
        plsc.subcore_barrier()
        edge_pass(y1_hbm, bidx, val2_hbm)
        plsc.subcore_barrier()
        all_stripes(writeback_stripe, y2_hbm, base)
        plsc.subcore_barrier()
        return carry

    lax.fori_loop(0, CPS, chunk, 0)


_spmm_sc_call = pl.kernel(
    _spmm_sc_body,
    out_type=[
        jax.ShapeDtypeStruct((B * N, F), jnp.float32),
        jax.ShapeDtypeStruct((B * N, F), jnp.float32),
    ],
    mesh=plsc.VectorSubcoreMesh(core_axis_name="c", subcore_axis_name="s"),
    compiler_params=pltpu.CompilerParams(needs_layout_passes=False),
    scratch_types=[
        pltpu.VMEM_SHARED((N, F), jnp.float32),
        pltpu.VMEM((SB, F), jnp.float32),
        [pltpu.VMEM((EW,), jnp.int32) for _ in range(NSLOT)],
        [pltpu.VMEM((EW,), jnp.int32) for _ in range(NSLOT)],
        [pltpu.VMEM((EW,), jnp.float32) for _ in range(NSLOT)],
        [pltpu.VMEM((EW, F), jnp.float32) for _ in range(NSLOT)],
        [pltpu.VMEM((EW, F), jnp.float32) for _ in range(NSLOT)],
        [pltpu.SemaphoreType.DMA for _ in range(NSLOT)],
        [pltpu.SemaphoreType.DMA for _ in range(NSLOT)],
        [pltpu.SemaphoreType.DMA for _ in range(NSLOT)],
    ],
)


def _spmm_pair(row, colg8, val, val2, x):
    y1, y2 = _spmm_sc_call(x.reshape(B * N, F), colg8, row, val, val2)
    return y1.reshape(B, N, F), y2.reshape(B, N, F)


def _prep_w(W, out_dim):
    Wk = W.reshape(U + 1, 3, out_dim).transpose(1, 0, 2)
    return jnp.pad(Wk, ((0, 0), (0, F - 1 - U), (0, 0)))


def kernel(inputs, hidden_state, edge_row, edge_col, lap_val, W_gate, b_gate,
           W_cand, b_cand, W_proj, b_proj):
    inp = inputs.reshape(B, N, 1)
    hx = hidden_state[0].reshape(B, N, U)
    x0 = jnp.concatenate([inp, hx, jnp.zeros((B, N, F - 1 - U), jnp.float32)], axis=2)

    wg = _prep_w(W_gate, GATE)
    wc = _prep_w(W_cand, U)

    colg8 = (edge_col[None, :]
             + (jnp.arange(B, dtype=jnp.int32) * N)[:, None]).reshape(-1)
    val2 = lap_val * 2.0

    x1, x2 = _spmm_pair(edge_row, colg8, lap_val, val2, x0)
    xc, u = _gate_call(
        x0.reshape(8 * N, F), x1.reshape(8 * N, F), x2.reshape(8 * N, F),
        hx.reshape(8 * N, U), wg, b_gate.reshape(1, GATE))

    xcb = xc.reshape(B, N, F)
    xc1, xc2 = _spmm_pair(edge_row, colg8, lap_val, val2, xcb)
    h, p = _cand_call(
        xc, xc1.reshape(8 * N, F), xc2.reshape(8 * N, F), u,
        hx.reshape(8 * N, U), wc, b_cand.reshape(1, U),
        W_proj, b_proj.reshape(1, 1))

    out = p.reshape(B, N)
    hidden = h.reshape(1, B, N * U)
    return (out, hidden)


# speedup vs baseline: 2.7833x; 1.4237x over previous
import functools

import jax
import jax.numpy as jnp
from jax import lax
from jax.experimental import pallas as pl
from jax.experimental.pallas import tpu as pltpu
from jax.experimental.pallas import tpu_sc as plsc

N = 10000
E = 320000
B = 8
U = 64
F = 128
GATE = 2 * U

BN = 2000

NC = 2
NS = 16
SROWS = 624
TAIL = N - NS * SROWS
SB = 48
EPT = E // NS
EW = 40
NWIN = EPT // EW
NSLOT = 4
NFS = 5
CPS = B // NC


def _gate_body(x0_ref, x1_ref, x2_ref, hx_ref, w_ref, b_ref, xc_ref, u_ref):
    acc = (
        jnp.dot(x0_ref[...], w_ref[0], preferred_element_type=jnp.float32)
        + jnp.dot(x1_ref[...], w_ref[1], preferred_element_type=jnp.float32)
        + jnp.dot(x2_ref[...], w_ref[2], preferred_element_type=jnp.float32)
        + b_ref[...]
    )
    v = jax.nn.sigmoid(acc)
    r = v[:, :U]
    u = v[:, U:]
    u_ref[...] = u
    rh = r * hx_ref[...]
    xc_ref[...] = jnp.concatenate(
        [x0_ref[:, 0:1], rh, jnp.zeros((BN, F - 1 - U), jnp.float32)], axis=1
    )


def _cand_body(x0_ref, x1_ref, x2_ref, u_ref, hx_ref, w_ref, b_ref, wp_ref,
               bp_ref, h_ref, p_ref):
    acc = (
        jnp.dot(x0_ref[...], w_ref[0], preferred_element_type=jnp.float32)
        + jnp.dot(x1_ref[...], w_ref[1], preferred_element_type=jnp.float32)
        + jnp.dot(x2_ref[...], w_ref[2], preferred_element_type=jnp.float32)
        + b_ref[...]
    )
    c = jnp.tanh(acc)
    u = u_ref[...]
    h = u * hx_ref[...] + (1.0 - u) * c
    h_ref[...] = h
    p_ref[...] = jnp.dot(h, wp_ref[...], preferred_element_type=jnp.float32) + bp_ref[...]


def _row_spec(width):
    return pl.BlockSpec((BN, width), lambda i: (i, 0))


def _full_spec(shape):
    return pl.BlockSpec(shape, lambda i: tuple(0 for _ in shape))


def _gate_call(x0, x1, x2, hx, w, b):
    grid = (8 * N // BN,)
    return pl.pallas_call(
        _gate_body,
        grid=grid,
        in_specs=[
            _row_spec(F), _row_spec(F), _row_spec(F), _row_spec(U),
            _full_spec((3, F, GATE)), _full_spec((1, GATE)),
        ],
        out_specs=[_row_spec(F), _row_spec(U)],
        out_shape=[
            jax.ShapeDtypeStruct((8 * N, F), jnp.float32),
            jax.ShapeDtypeStruct((8 * N, U), jnp.float32),
        ],
    )(x0, x1, x2, hx, w, b)


def _cand_call(x0, x1, x2, u, hx, w, b, wp, bp):
    grid = (8 * N // BN,)
    return pl.pallas_call(
        _cand_body,
        grid=grid,
        in_specs=[
            _row_spec(F), _row_spec(F), _row_spec(F), _row_spec(U), _row_spec(U),
            _full_spec((3, F, U)), _full_spec((1, U)),
            _full_spec((U, 1)), _full_spec((1, 1)),
        ],
        out_specs=[_row_spec(U), _row_spec(1)],
        out_shape=[
            jax.ShapeDtypeStruct((8 * N, U), jnp.float32),
            jax.ShapeDtypeStruct((8 * N, 1), jnp.float32),
        ],
    )(x0, x1, x2, u, hx, w, b, wp, bp)


def _spmm_sc_body(x_hbm, colg_hbm, row_hbm, val_hbm, val2_hbm, y1_hbm, y2_hbm,
                  acc, sbuf, colbs, rowbs, valbs, rowsbs, sclbs,
                  isems, gsems, ssems):
    ci = lax.axis_index("c")
    si = lax.axis_index("s")
    t0 = si * SROWS
    e0 = si * EPT
    nf = F // 16

    for s_init in range(NSLOT):
        def pad_row(rr, c2, _sb=sclbs[s_init]):
            for jj in range(NFS, nf):
                _sb[rr, pl.ds(jj * 16, 16)] = jnp.zeros((16,), jnp.float32)
            return c2

        lax.fori_loop(0, EW, pad_row, 0)

    def edge_pass(src_hbm, bidx, vhbm):
        cbase = bidx * E + e0

        def in_args(w, s):
            return (
                (colg_hbm.at[pl.ds(cbase + w * EW, EW)], colbs[s], isems[s]),
                (row_hbm.at[pl.ds(e0 + w * EW, EW)], rowbs[s], isems[s]),
                (vhbm.at[pl.ds((e0 + w * EW) * 16, EW * 16)], valbs[s], isems[s]),
            )

        def issue_inputs(w, s):
            for a in in_args(w, s):
                pltpu.async_copy(*a)

        def wait_inputs(w, s):
            for a in in_args(w, s):
                pltpu.make_async_copy(*a).wait()

        def gather_start(s):
            pltpu.async_copy(src_hbm.at[colbs[s]], rowsbs[s], gsems[s])

        def gather_wait(s):
            pltpu.make_async_copy(src_hbm.at[colbs[s]], rowsbs[s], gsems[s]).wait()

        def scatter_start(s):
            pltpu.async_copy(sclbs[s], acc.at[rowbs[s]], ssems[s], add=True)

        def scatter_wait(s):
            pltpu.make_async_copy(sclbs[s], acc.at[rowbs[s]], ssems[s]).wait()

        def scale(s):
            rb, sb_, vb = rowsbs[s], sclbs[s], valbs[s]

            def scl(k2, c2):
                for uu in range(2):
                    k = k2 * 2 + uu
                    v = vb[pl.ds(k * 16, 16)]
                    for jj in range(NFS):
                        sl = pl.ds(jj * 16, 16)
                        sb_[k, sl] = rb[k, sl] * v
                return c2

            lax.fori_loop(0, EW // 2, scl, 0)

        issue_inputs(0, 0)
        issue_inputs(1, 1)
        wait_inputs(0, 0)
        gather_start(0)

        def quad(q, carry):
            w0 = q * NSLOT
            for s_ in range(NSLOT):
                w = w0 + s_
                sp1 = (s_ + 1) % NSLOT
                sp2 = (s_ + 2) % NSLOT

                @pl.when(w >= 2)
                def _():
                    scatter_wait(sp2)

                @pl.when(w + 2 < NWIN)
                def _():
                    issue_inputs(w + 2, sp2)

                @pl.when(w + 1 < NWIN)
                def _():
                    wait_inputs(w + 1, sp1)
                    gather_start(sp1)

                gather_wait(s_)
                scale(s_)
                scatter_start(s_)
            return carry

        lax.fori_loop(0, NWIN // NSLOT, quad, 0)
        scatter_wait((NWIN - 2) % NSLOT)
        scatter_wait((NWIN - 1) % NSLOT)

    def zero_stripe(off, size):
        def zero_row(rr, c2):
            for jj in range(nf):
                sbuf[rr, pl.ds(jj * 16, 16)] = jnp.zeros((16,), jnp.float32)
            return c2

        lax.fori_loop(0, min(size, SB), zero_row, 0)

        def blk(i, c2):
            pltpu.sync_copy(sbuf.at[pl.ds(0, min(size, SB))],
                            acc.at[pl.ds(off + i * SB, min(size, SB))])
            return c2

        lax.fori_loop(0, pl.cdiv(size, SB), blk, 0)

    def negx_stripe(base, off, size):
        bs = min(size, SB)

        def blk(i, c2):
            pltpu.sync_copy(x_hbm.at[pl.ds(base + off + i * SB, bs)],
                            sbuf.at[pl.ds(0, bs)])

            def neg_row(rr, c3):
                for jj in range(nf):
                    sl = pl.ds(jj * 16, 16)
                    sbuf[rr, sl] = -sbuf[rr, sl]
                return c3

            lax.fori_loop(0, bs, neg_row, 0)
            pltpu.sync_copy(sbuf.at[pl.ds(0, bs)], acc.at[pl.ds(off + i * SB, bs)])
            return c2

        lax.fori_loop(0, pl.cdiv(size, SB), blk, 0)

    def writeback_stripe(y_hbm, base, off, size):
        pltpu.sync_copy(acc.at[pl.ds(off, size)], y_hbm.at[pl.ds(base + off, size)])

    def all_stripes(fn, *args):
        fn(*args, t0, SROWS)

        @pl.when(si == 0)
        def _():
            fn(*args, NS * SROWS, TAIL)

    def chunk(j, carry):
        bidx = ci * CPS + j
        base = bidx * N

        all_stripes(zero_stripe)
        plsc.subcore_barrier()
        edge_pass(x_hbm, bidx, val_hbm)
        plsc.subcore_barrier()
        all_stripes(writeback_stripe, y1_hbm, base)
        plsc.subcore_barrier()

        all_stripes(negx_stripe, base)
        plsc.subcore_barrier()
        edge_pass(y1_hbm, bidx, val2_hbm)
        plsc.subcore_barrier()
        all_stripes(writeback_stripe, y2_hbm, base)
        plsc.subcore_barrier()
        return carry

    lax.fori_loop(0, CPS, chunk, 0)


_spmm_sc_call = pl.kernel(
    _spmm_sc_body,
    out_type=[
        jax.ShapeDtypeStruct((B * N, F), jnp.float32),
        jax.ShapeDtypeStruct((B * N, F), jnp.float32),
    ],
    mesh=plsc.VectorSubcoreMesh(core_axis_name="c", subcore_axis_name="s"),
    compiler_params=pltpu.CompilerParams(needs_layout_passes=False),
    scratch_types=[
        pltpu.VMEM_SHARED((N, F), jnp.float32),
        pltpu.VMEM((SB, F), jnp.float32),
        [pltpu.VMEM((EW,), jnp.int32) for _ in range(NSLOT)],
        [pltpu.VMEM((EW,), jnp.int32) for _ in range(NSLOT)],
        [pltpu.VMEM((EW * 16,), jnp.float32) for _ in range(NSLOT)],
        [pltpu.VMEM((EW, F), jnp.float32) for _ in range(NSLOT)],
        [pltpu.VMEM((EW, F), jnp.float32) for _ in range(NSLOT)],
        [pltpu.SemaphoreType.DMA for _ in range(NSLOT)],
        [pltpu.SemaphoreType.DMA for _ in range(NSLOT)],
        [pltpu.SemaphoreType.DMA for _ in range(NSLOT)],
    ],
)


def _spmm_pair(row, colg8, val, val2, x):
    y1, y2 = _spmm_sc_call(x.reshape(B * N, F), colg8, row, val, val2)
    return y1.reshape(B, N, F), y2.reshape(B, N, F)


def _prep_w(W, out_dim):
    Wk = W.reshape(U + 1, 3, out_dim).transpose(1, 0, 2)
    return jnp.pad(Wk, ((0, 0), (0, F - 1 - U), (0, 0)))


def kernel(inputs, hidden_state, edge_row, edge_col, lap_val, W_gate, b_gate,
           W_cand, b_cand, W_proj, b_proj):
    inp = inputs.reshape(B, N, 1)
    hx = hidden_state[0].reshape(B, N, U)
    x0 = jnp.concatenate([inp, hx, jnp.zeros((B, N, F - 1 - U), jnp.float32)], axis=2)

    wg = _prep_w(W_gate, GATE)
    wc = _prep_w(W_cand, U)

    colg8 = (edge_col[None, :]
             + (jnp.arange(B, dtype=jnp.int32) * N)[:, None]).reshape(-1)
    valx = jnp.broadcast_to(lap_val[:, None], (E, 16)).reshape(E * 16)
    val2x = 2.0 * valx

    x1, x2 = _spmm_pair(edge_row, colg8, valx, val2x, x0)
    xc, u = _gate_call(
        x0.reshape(8 * N, F), x1.reshape(8 * N, F), x2.reshape(8 * N, F),
        hx.reshape(8 * N, U), wg, b_gate.reshape(1, GATE))

    xcb = xc.reshape(B, N, F)
    xc1, xc2 = _spmm_pair(edge_row, colg8, valx, val2x, xcb)
    h, p = _cand_call(
        xc, xc1.reshape(8 * N, F), xc2.reshape(8 * N, F), u,
        hx.reshape(8 * N, U), wc, b_cand.reshape(1, U),
        W_proj, b_proj.reshape(1, 1))

    out = p.reshape(B, N)
    hidden = h.reshape(1, B, N * U)
    return (out, hidden)


# speedup vs baseline: 2.9709x; 1.0674x over previous
import functools

import jax
import jax.numpy as jnp
from jax import lax
from jax.experimental import pallas as pl
from jax.experimental.pallas import tpu as pltpu
from jax.experimental.pallas import tpu_sc as plsc

N = 10000
E = 320000
B = 8
U = 64
F = 128
GATE = 2 * U

BN = 2000

NC = 2
NS = 16
SROWS = 624
TAIL = N - NS * SROWS
SB = 16
EPT = E // NS
EW = 32
NWIN = EPT // EW
NSLOT = 5
NFS = 5
CPS = B // NC


def _gate_body(x0_ref, x1_ref, x2_ref, hx_ref, w_ref, b_ref, xc_ref, u_ref):
    acc = (
        jnp.dot(x0_ref[...], w_ref[0], preferred_element_type=jnp.float32)
        + jnp.dot(x1_ref[...], w_ref[1], preferred_element_type=jnp.float32)
        + jnp.dot(x2_ref[...], w_ref[2], preferred_element_type=jnp.float32)
        + b_ref[...]
    )
    v = jax.nn.sigmoid(acc)
    r = v[:, :U]
    u = v[:, U:]
    u_ref[...] = u
    rh = r * hx_ref[...]
    xc_ref[...] = jnp.concatenate(
        [x0_ref[:, 0:1], rh, jnp.zeros((BN, F - 1 - U), jnp.float32)], axis=1
    )


def _cand_body(x0_ref, x1_ref, x2_ref, u_ref, hx_ref, w_ref, b_ref, wp_ref,
               bp_ref, h_ref, p_ref):
    acc = (
        jnp.dot(x0_ref[...], w_ref[0], preferred_element_type=jnp.float32)
        + jnp.dot(x1_ref[...], w_ref[1], preferred_element_type=jnp.float32)
        + jnp.dot(x2_ref[...], w_ref[2], preferred_element_type=jnp.float32)
        + b_ref[...]
    )
    c = jnp.tanh(acc)
    u = u_ref[...]
    h = u * hx_ref[...] + (1.0 - u) * c
    h_ref[...] = h
    p_ref[...] = jnp.dot(h, wp_ref[...], preferred_element_type=jnp.float32) + bp_ref[...]


def _row_spec(width):
    return pl.BlockSpec((BN, width), lambda i: (i, 0))


def _full_spec(shape):
    return pl.BlockSpec(shape, lambda i: tuple(0 for _ in shape))


def _gate_call(x0, x1, x2, hx, w, b):
    grid = (8 * N // BN,)
    return pl.pallas_call(
        _gate_body,
        grid=grid,
        in_specs=[
            _row_spec(F), _row_spec(F), _row_spec(F), _row_spec(U),
            _full_spec((3, F, GATE)), _full_spec((1, GATE)),
        ],
        out_specs=[_row_spec(F), _row_spec(U)],
        out_shape=[
            jax.ShapeDtypeStruct((8 * N, F), jnp.float32),
            jax.ShapeDtypeStruct((8 * N, U), jnp.float32),
        ],
    )(x0, x1, x2, hx, w, b)


def _cand_call(x0, x1, x2, u, hx, w, b, wp, bp):
    grid = (8 * N // BN,)
    return pl.pallas_call(
        _cand_body,
        grid=grid,
        in_specs=[
            _row_spec(F), _row_spec(F), _row_spec(F), _row_spec(U), _row_spec(U),
            _full_spec((3, F, U)), _full_spec((1, U)),
            _full_spec((U, 1)), _full_spec((1, 1)),
        ],
        out_specs=[_row_spec(U), _row_spec(1)],
        out_shape=[
            jax.ShapeDtypeStruct((8 * N, U), jnp.float32),
            jax.ShapeDtypeStruct((8 * N, 1), jnp.float32),
        ],
    )(x0, x1, x2, u, hx, w, b, wp, bp)


def _spmm_sc_body(x_hbm, colg_hbm, row_hbm, val_hbm, val2_hbm, y1_hbm, y2_hbm,
                  acc, sbuf, colbs, rowbs, valbs, rowsbs, sclbs,
                  isems, gsems, ssems):
    ci = lax.axis_index("c")
    si = lax.axis_index("s")
    t0 = si * SROWS
    e0 = si * EPT
    nf = F // 16

    for s_init in range(NSLOT):
        def pad_row(rr, c2, _sb=sclbs[s_init]):
            for jj in range(NFS, nf):
                _sb[rr, pl.ds(jj * 16, 16)] = jnp.zeros((16,), jnp.float32)
            return c2

        lax.fori_loop(0, EW, pad_row, 0)

    def edge_pass(src_hbm, bidx, vhbm):
        cbase = bidx * E + e0

        def in_args(w, s):
            return (
                (colg_hbm.at[pl.ds(cbase + w * EW, EW)], colbs[s], isems[s]),
                (row_hbm.at[pl.ds(e0 + w * EW, EW)], rowbs[s], isems[s]),
                (vhbm.at[pl.ds((e0 + w * EW) * 16, EW * 16)], valbs[s], isems[s]),
            )

        def issue_inputs(w, s):
            for a in in_args(w, s):
                pltpu.async_copy(*a)

        def wait_inputs(w, s):
            for a in in_args(w, s):
                pltpu.make_async_copy(*a).wait()

        def gather_start(s):
            pltpu.async_copy(src_hbm.at[colbs[s]], rowsbs[s], gsems[s])

        def gather_wait(s):
            pltpu.make_async_copy(src_hbm.at[colbs[s]], rowsbs[s], gsems[s]).wait()

        def scatter_start(s):
            pltpu.async_copy(sclbs[s], acc.at[rowbs[s]], ssems[s], add=True)

        def scatter_wait(s):
            pltpu.make_async_copy(sclbs[s], acc.at[rowbs[s]], ssems[s]).wait()

        def scale(s):
            rb, sb_, vb = rowsbs[s], sclbs[s], valbs[s]

            def scl(k2, c2):
                for uu in range(2):
                    k = k2 * 2 + uu
                    v = vb[pl.ds(k * 16, 16)]
                    for jj in range(NFS):
                        sl = pl.ds(jj * 16, 16)
                        sb_[k, sl] = rb[k, sl] * v
                return c2

            lax.fori_loop(0, EW // 2, scl, 0)

        issue_inputs(0, 0)
        issue_inputs(1, 1)
        issue_inputs(2, 2)
        wait_inputs(0, 0)
        gather_start(0)
        wait_inputs(1, 1)
        gather_start(1)

        def quint(q, carry):
            w0 = q * NSLOT
            for s_ in range(NSLOT):
                w = w0 + s_
                sp2 = (s_ + 2) % NSLOT
                sp3 = (s_ + 3) % NSLOT

                @pl.when(w >= 2)
                def _():
                    scatter_wait(sp3)

                @pl.when(w + 3 < NWIN)
                def _():
                    issue_inputs(w + 3, sp3)

                @pl.when(w + 2 < NWIN)
                def _():
                    wait_inputs(w + 2, sp2)
                    gather_start(sp2)

                gather_wait(s_)
                scale(s_)
                scatter_start(s_)
            return carry

        lax.fori_loop(0, NWIN // NSLOT, quint, 0)
        scatter_wait((NWIN - 2) % NSLOT)
        scatter_wait((NWIN - 1) % NSLOT)

    def zero_stripe(off, size):
        def zero_row(rr, c2):
            for jj in range(nf):
                sbuf[rr, pl.ds(jj * 16, 16)] = jnp.zeros((16,), jnp.float32)
            return c2

        lax.fori_loop(0, min(size, SB), zero_row, 0)

        def blk(i, c2):
            pltpu.sync_copy(sbuf.at[pl.ds(0, min(size, SB))],
                            acc.at[pl.ds(off + i * SB, min(size, SB))])
            return c2

        lax.fori_loop(0, pl.cdiv(size, SB), blk, 0)

    def negx_stripe(base, off, size):
        bs = min(size, SB)

        def blk(i, c2):
            pltpu.sync_copy(x_hbm.at[pl.ds(base + off + i * SB, bs)],
                            sbuf.at[pl.ds(0, bs)])

            def neg_row(rr, c3):
                for jj in range(nf):
                    sl = pl.ds(jj * 16, 16)
                    sbuf[rr, sl] = -sbuf[rr, sl]
                return c3

            lax.fori_loop(0, bs, neg_row, 0)
            pltpu.sync_copy(sbuf.at[pl.ds(0, bs)], acc.at[pl.ds(off + i * SB, bs)])
            return c2

        lax.fori_loop(0, pl.cdiv(size, SB), blk, 0)

    def writeback_stripe(y_hbm, base, off, size):
        pltpu.sync_copy(acc.at[pl.ds(off, size)], y_hbm.at[pl.ds(base + off, size)])

    def all_stripes(fn, *args):
        fn(*args, t0, SROWS)

        @pl.when(si == 0)
        def _():
            fn(*args, NS * SROWS, TAIL)

    def chunk(j, carry):
        bidx = ci * CPS + j
        base = bidx * N

        all_stripes(zero_stripe)
        plsc.subcore_barrier()
        edge_pass(x_hbm, bidx, val_hbm)
        plsc.subcore_barrier()
        all_stripes(writeback_stripe, y1_hbm, base)
        plsc.subcore_barrier()

        all_stripes(negx_stripe, base)
        plsc.subcore_barrier()
        edge_pass(y1_hbm, bidx, val2_hbm)
        plsc.subcore_barrier()
        all_stripes(writeback_stripe, y2_hbm, base)
        plsc.subcore_barrier()
        return carry

    lax.fori_loop(0, CPS, chunk, 0)


_spmm_sc_call = pl.kernel(
    _spmm_sc_body,
    out_type=[
        jax.ShapeDtypeStruct((B * N, F), jnp.float32),
        jax.ShapeDtypeStruct((B * N, F), jnp.float32),
    ],
    mesh=plsc.VectorSubcoreMesh(core_axis_name="c", subcore_axis_name="s"),
    compiler_params=pltpu.CompilerParams(needs_layout_passes=False),
    scratch_types=[
        pltpu.VMEM_SHARED((N, F), jnp.float32),
        pltpu.VMEM((SB, F), jnp.float32),
        [pltpu.VMEM((EW,), jnp.int32) for _ in range(NSLOT)],
        [pltpu.VMEM((EW,), jnp.int32) for _ in range(NSLOT)],
        [pltpu.VMEM((EW * 16,), jnp.float32) for _ in range(NSLOT)],
        [pltpu.VMEM((EW, F), jnp.float32) for _ in range(NSLOT)],
        [pltpu.VMEM((EW, F), jnp.float32) for _ in range(NSLOT)],
        [pltpu.SemaphoreType.DMA for _ in range(NSLOT)],
        [pltpu.SemaphoreType.DMA for _ in range(NSLOT)],
        [pltpu.SemaphoreType.DMA for _ in range(NSLOT)],
    ],
)


def _spmm_pair(row, colg8, val, val2, x):
    y1, y2 = _spmm_sc_call(x.reshape(B * N, F), colg8, row, val, val2)
    return y1.reshape(B, N, F), y2.reshape(B, N, F)


def _prep_w(W, out_dim):
    Wk = W.reshape(U + 1, 3, out_dim).transpose(1, 0, 2)
    return jnp.pad(Wk, ((0, 0), (0, F - 1 - U), (0, 0)))


def kernel(inputs, hidden_state, edge_row, edge_col, lap_val, W_gate, b_gate,
           W_cand, b_cand, W_proj, b_proj):
    inp = inputs.reshape(B, N, 1)
    hx = hidden_state[0].reshape(B, N, U)
    x0 = jnp.concatenate([inp, hx, jnp.zeros((B, N, F - 1 - U), jnp.float32)], axis=2)

    wg = _prep_w(W_gate, GATE)
    wc = _prep_w(W_cand, U)

    colg8 = (edge_col[None, :]
             + (jnp.arange(B, dtype=jnp.int32) * N)[:, None]).reshape(-1)
    valx = jnp.broadcast_to(lap_val[:, None], (E, 16)).reshape(E * 16)
    val2x = 2.0 * valx

    x1, x2 = _spmm_pair(edge_row, colg8, valx, val2x, x0)
    xc, u = _gate_call(
        x0.reshape(8 * N, F), x1.reshape(8 * N, F), x2.reshape(8 * N, F),
        hx.reshape(8 * N, U), wg, b_gate.reshape(1, GATE))

    xcb = xc.reshape(B, N, F)
    xc1, xc2 = _spmm_pair(edge_row, colg8, valx, val2x, xcb)
    h, p = _cand_call(
        xc, xc1.reshape(8 * N, F), xc2.reshape(8 * N, F), u,
        hx.reshape(8 * N, U), wc, b_cand.reshape(1, U),
        W_proj, b_proj.reshape(1, 1))

    out = p.reshape(B, N)
    hidden = h.reshape(1, B, N * U)
    return (out, hidden)


# speedup vs baseline: 3.2093x; 1.0803x over previous
import functools

import jax
import jax.numpy as jnp
from jax import lax
from jax.experimental import pallas as pl
from jax.experimental.pallas import tpu as pltpu
from jax.experimental.pallas import tpu_sc as plsc

N = 10000
E = 320000
B = 8
U = 64
F = 128
GATE = 2 * U

BN = 2000

NC = 2
NS = 16
SROWS = 624
TAIL = N - NS * SROWS
SB = 16
EPT = E // NS
EW = 40
NWIN = EPT // EW
NSLOT = 5
NFS = 5
CPS = B // NC


def _gate_body(x0_ref, x1_ref, x2_ref, hx_ref, w_ref, b_ref, xc_ref, u_ref):
    acc = (
        jnp.dot(x0_ref[...], w_ref[0], preferred_element_type=jnp.float32)
        + jnp.dot(x1_ref[...], w_ref[1], preferred_element_type=jnp.float32)
        + jnp.dot(x2_ref[...], w_ref[2], preferred_element_type=jnp.float32)
        + b_ref[...]
    )
    v = jax.nn.sigmoid(acc)
    r = v[:, :U]
    u = v[:, U:]
    u_ref[...] = u
    rh = r * hx_ref[...]
    xc_ref[...] = jnp.concatenate(
        [x0_ref[:, 0:1], rh, jnp.zeros((BN, F - 1 - U), jnp.float32)], axis=1
    )


def _cand_body(x0_ref, x1_ref, x2_ref, u_ref, hx_ref, w_ref, b_ref, wp_ref,
               bp_ref, h_ref, p_ref):
    acc = (
        jnp.dot(x0_ref[...], w_ref[0], preferred_element_type=jnp.float32)
        + jnp.dot(x1_ref[...], w_ref[1], preferred_element_type=jnp.float32)
        + jnp.dot(x2_ref[...], w_ref[2], preferred_element_type=jnp.float32)
        + b_ref[...]
    )
    c = jnp.tanh(acc)
    u = u_ref[...]
    h = u * hx_ref[...] + (1.0 - u) * c
    h_ref[...] = h
    p_ref[...] = jnp.dot(h, wp_ref[...], preferred_element_type=jnp.float32) + bp_ref[...]


def _row_spec(width):
    return pl.BlockSpec((BN, width), lambda i: (i, 0))


def _full_spec(shape):
    return pl.BlockSpec(shape, lambda i: tuple(0 for _ in shape))


def _gate_call(x0, x1, x2, hx, w, b):
    grid = (8 * N // BN,)
    return pl.pallas_call(
        _gate_body,
        grid=grid,
        in_specs=[
            _row_spec(F), _row_spec(F), _row_spec(F), _row_spec(U),
            _full_spec((3, F, GATE)), _full_spec((1, GATE)),
        ],
        out_specs=[_row_spec(F), _row_spec(U)],
        out_shape=[
            jax.ShapeDtypeStruct((8 * N, F), jnp.float32),
            jax.ShapeDtypeStruct((8 * N, U), jnp.float32),
        ],
    )(x0, x1, x2, hx, w, b)


def _cand_call(x0, x1, x2, u, hx, w, b, wp, bp):
    grid = (8 * N // BN,)
    return pl.pallas_call(
        _cand_body,
        grid=grid,
        in_specs=[
            _row_spec(F), _row_spec(F), _row_spec(F), _row_spec(U), _row_spec(U),
            _full_spec((3, F, U)), _full_spec((1, U)),
            _full_spec((U, 1)), _full_spec((1, 1)),
        ],
        out_specs=[_row_spec(U), _row_spec(1)],
        out_shape=[
            jax.ShapeDtypeStruct((8 * N, U), jnp.float32),
            jax.ShapeDtypeStruct((8 * N, 1), jnp.float32),
        ],
    )(x0, x1, x2, u, hx, w, b, wp, bp)


def _spmm_sc_body(x_hbm, colg_hbm, row_hbm, val_hbm, y1_hbm, y2_hbm,
                  acc, sbuf, colbs, rowbs, valbs, rowsbs,
                  isems, gsems, ssems):
    ci = lax.axis_index("c")
    si = lax.axis_index("s")
    t0 = si * SROWS
    e0 = si * EPT
    nf = F // 16

    def edge_pass(src_hbm, bidx, dbl):
        cbase = bidx * E + e0

        def in_args(w, s):
            return (
                (colg_hbm.at[pl.ds(cbase + w * EW, EW)], colbs[s], isems[s]),
                (row_hbm.at[pl.ds(e0 + w * EW, EW)], rowbs[s], isems[s]),
                (val_hbm.at[pl.ds((e0 + w * EW) * 16, EW * 16)], valbs[s], isems[s]),
            )

        def issue_inputs(w, s):
            for a in in_args(w, s):
                pltpu.async_copy(*a)

        def wait_inputs(w, s):
            for a in in_args(w, s):
                pltpu.make_async_copy(*a).wait()

        def gather_start(s):
            pltpu.async_copy(src_hbm.at[colbs[s]], rowsbs[s], gsems[s])

        def gather_wait(s):
            pltpu.make_async_copy(src_hbm.at[colbs[s]], rowsbs[s], gsems[s]).wait()

        def scatter_start(s):
            pltpu.async_copy(rowsbs[s], acc.at[rowbs[s]], ssems[s], add=True)

        def scatter_wait(s):
            pltpu.make_async_copy(rowsbs[s], acc.at[rowbs[s]], ssems[s]).wait()

        def scale(s):
            rb, vb = rowsbs[s], valbs[s]

            def scl(k2, c2):
                for uu in range(2):
                    k = k2 * 2 + uu
                    v = vb[pl.ds(k * 16, 16)]
                    if dbl:
                        v = v + v
                    for jj in range(NFS):
                        sl = pl.ds(jj * 16, 16)
                        rb[k, sl] = rb[k, sl] * v
                return c2

            lax.fori_loop(0, EW // 2, scl, 0)

        issue_inputs(0, 0)
        issue_inputs(1, 1)
        issue_inputs(2, 2)
        wait_inputs(0, 0)
        gather_start(0)
        wait_inputs(1, 1)
        gather_start(1)

        def quint(q, carry):
            w0 = q * NSLOT
            for s_ in range(NSLOT):
                w = w0 + s_
                sp2 = (s_ + 2) % NSLOT
                sp3 = (s_ + 3) % NSLOT

                @pl.when(w >= 2)
                def _():
                    scatter_wait(sp3)

                @pl.when(w + 3 < NWIN)
                def _():
                    issue_inputs(w + 3, sp3)

                @pl.when(w + 2 < NWIN)
                def _():
                    wait_inputs(w + 2, sp2)
                    gather_start(sp2)

                gather_wait(s_)
                scale(s_)
                scatter_start(s_)
            return carry

        lax.fori_loop(0, NWIN // NSLOT, quint, 0)
        scatter_wait((NWIN - 2) % NSLOT)
        scatter_wait((NWIN - 1) % NSLOT)

    def zero_stripe(off, size):
        def zero_row(rr, c2):
            for jj in range(nf):
                sbuf[rr, pl.ds(jj * 16, 16)] = jnp.zeros((16,), jnp.float32)
            return c2

        lax.fori_loop(0, min(size, SB), zero_row, 0)

        def blk(i, c2):
            pltpu.sync_copy(sbuf.at[pl.ds(0, min(size, SB))],
                            acc.at[pl.ds(off + i * SB, min(size, SB))])
            return c2

        lax.fori_loop(0, pl.cdiv(size, SB), blk, 0)

    def negx_stripe(base, off, size):
        bs = min(size, SB)

        def blk(i, c2):
            pltpu.sync_copy(x_hbm.at[pl.ds(base + off + i * SB, bs)],
                            sbuf.at[pl.ds(0, bs)])

            def neg_row(rr, c3):
                for jj in range(nf):
                    sl = pl.ds(jj * 16, 16)
                    sbuf[rr, sl] = -sbuf[rr, sl]
                return c3

            lax.fori_loop(0, bs, neg_row, 0)
            pltpu.sync_copy(sbuf.at[pl.ds(0, bs)], acc.at[pl.ds(off + i * SB, bs)])
            return c2

        lax.fori_loop(0, pl.cdiv(size, SB), blk, 0)

    def writeback_stripe(y_hbm, base, off, size):
        pltpu.sync_copy(acc.at[pl.ds(off, size)], y_hbm.at[pl.ds(base + off, size)])

    def all_stripes(fn, *args):
        fn(*args, t0, SROWS)

        @pl.when(si == 0)
        def _():
            fn(*args, NS * SROWS, TAIL)

    def chunk(j, carry):
        bidx = ci * CPS + j
        base = bidx * N

        all_stripes(zero_stripe)
        plsc.subcore_barrier()
        edge_pass(x_hbm, bidx, False)
        plsc.subcore_barrier()
        all_stripes(writeback_stripe, y1_hbm, base)
        plsc.subcore_barrier()

        all_stripes(negx_stripe, base)
        plsc.subcore_barrier()
        edge_pass(y1_hbm, bidx, True)
        plsc.subcore_barrier()
        all_stripes(writeback_stripe, y2_hbm, base)
        plsc.subcore_barrier()
        return carry

    lax.fori_loop(0, CPS, chunk, 0)


_spmm_sc_call = pl.kernel(
    _spmm_sc_body,
    out_type=[
        jax.ShapeDtypeStruct((B * N, F), jnp.float32),
        jax.ShapeDtypeStruct((B * N, F), jnp.float32),
    ],
    mesh=plsc.VectorSubcoreMesh(core_axis_name="c", subcore_axis_name="s"),
    compiler_params=pltpu.CompilerParams(needs_layout_passes=False),
    scratch_types=[
        pltpu.VMEM_SHARED((N, F), jnp.float32),
        pltpu.VMEM((SB, F), jnp.float32),
        [pltpu.VMEM((EW,), jnp.int32) for _ in range(NSLOT)],
        [pltpu.VMEM((EW,), jnp.int32) for _ in range(NSLOT)],
        [pltpu.VMEM((EW * 16,), jnp.float32) for _ in range(NSLOT)],
        [pltpu.VMEM((EW, F), jnp.float32) for _ in range(NSLOT)],
        [pltpu.SemaphoreType.DMA for _ in range(NSLOT)],
        [pltpu.SemaphoreType.DMA for _ in range(NSLOT)],
        [pltpu.SemaphoreType.DMA for _ in range(NSLOT)],
    ],
)


def _spmm_pair(row, colg8, val, x):
    y1, y2 = _spmm_sc_call(x.reshape(B * N, F), colg8, row, val)
    return y1.reshape(B, N, F), y2.reshape(B, N, F)


def _prep_w(W, out_dim):
    Wk = W.reshape(U + 1, 3, out_dim).transpose(1, 0, 2)
    return jnp.pad(Wk, ((0, 0), (0, F - 1 - U), (0, 0)))


def kernel(inputs, hidden_state, edge_row, edge_col, lap_val, W_gate, b_gate,
           W_cand, b_cand, W_proj, b_proj):
    inp = inputs.reshape(B, N, 1)
    hx = hidden_state[0].reshape(B, N, U)
    x0 = jnp.concatenate([inp, hx, jnp.zeros((B, N, F - 1 - U), jnp.float32)], axis=2)

    wg = _prep_w(W_gate, GATE)
    wc = _prep_w(W_cand, U)

    colg8 = (edge_col[None, :]
             + (jnp.arange(B, dtype=jnp.int32) * N)[:, None]).reshape(-1)
    valx = jnp.broadcast_to(lap_val[:, None], (E, 16)).reshape(E * 16)

    x1, x2 = _spmm_pair(edge_row, colg8, valx, x0)
    xc, u = _gate_call(
        x0.reshape(8 * N, F), x1.reshape(8 * N, F), x2.reshape(8 * N, F),
        hx.reshape(8 * N, U), wg, b_gate.reshape(1, GATE))

    xcb = xc.reshape(B, N, F)
    xc1, xc2 = _spmm_pair(edge_row, colg8, valx, xcb)
    h, p = _cand_call(
        xc, xc1.reshape(8 * N, F), xc2.reshape(8 * N, F), u,
        hx.reshape(8 * N, U), wc, b_cand.reshape(1, U),
        W_proj, b_proj.reshape(1, 1))

    out = p.reshape(B, N)
    hidden = h.reshape(1, B, N * U)
    return (out, hidden)


# speedup vs baseline: 3.3046x; 1.0297x over previous
import functools

import jax
import jax.numpy as jnp
from jax import lax
from jax.experimental import pallas as pl
from jax.experimental.pallas import tpu as pltpu
from jax.experimental.pallas import tpu_sc as plsc

N = 10000
E = 320000
B = 8
U = 64
F = 128
GATE = 2 * U

BN = 2000

NC = 2
NS = 16
SROWS = 624
TAIL = N - NS * SROWS
SB = 16
EPT = E // NS
EW = 40
NWIN = EPT // EW
NSLOT = 5
NFS = 5
CPS = B // NC


def _gate_body(x0_ref, x1_ref, x2_ref, hx_ref, w_ref, b_ref, xc_ref, u_ref):
    acc = (
        jnp.dot(x0_ref[...], w_ref[0], preferred_element_type=jnp.float32)
        + jnp.dot(x1_ref[...], w_ref[1], preferred_element_type=jnp.float32)
        + jnp.dot(x2_ref[...], w_ref[2], preferred_element_type=jnp.float32)
        + b_ref[...]
    )
    v = jax.nn.sigmoid(acc)
    r = v[:, :U]
    u = v[:, U:]
    u_ref[...] = u
    rh = r * hx_ref[...]
    xc_ref[...] = jnp.concatenate(
        [x0_ref[:, 0:1], rh, jnp.zeros((BN, F - 1 - U), jnp.float32)], axis=1
    )


def _cand_body(x0_ref, x1_ref, x2_ref, u_ref, hx_ref, w_ref, b_ref, wp_ref,
               bp_ref, h_ref, p_ref):
    acc = (
        jnp.dot(x0_ref[...], w_ref[0], preferred_element_type=jnp.float32)
        + jnp.dot(x1_ref[...], w_ref[1], preferred_element_type=jnp.float32)
        + jnp.dot(x2_ref[...], w_ref[2], preferred_element_type=jnp.float32)
        + b_ref[...]
    )
    c = jnp.tanh(acc)
    u = u_ref[...]
    h = u * hx_ref[...] + (1.0 - u) * c
    h_ref[...] = h
    p_ref[...] = jnp.dot(h, wp_ref[...], preferred_element_type=jnp.float32) + bp_ref[...]


def _row_spec(width):
    return pl.BlockSpec((BN, width), lambda i: (i, 0))


def _full_spec(shape):
    return pl.BlockSpec(shape, lambda i: tuple(0 for _ in shape))


def _gate_call(x0, x1, x2, hx, w, b):
    grid = (8 * N // BN,)
    return pl.pallas_call(
        _gate_body,
        grid=grid,
        in_specs=[
            _row_spec(F), _row_spec(F), _row_spec(F), _row_spec(U),
            _full_spec((3, F, GATE)), _full_spec((1, GATE)),
        ],
        out_specs=[_row_spec(F), _row_spec(U)],
        out_shape=[
            jax.ShapeDtypeStruct((8 * N, F), jnp.float32),
            jax.ShapeDtypeStruct((8 * N, U), jnp.float32),
        ],
    )(x0, x1, x2, hx, w, b)


def _cand_call(x0, x1, x2, u, hx, w, b, wp, bp):
    grid = (8 * N // BN,)
    return pl.pallas_call(
        _cand_body,
        grid=grid,
        in_specs=[
            _row_spec(F), _row_spec(F), _row_spec(F), _row_spec(U), _row_spec(U),
            _full_spec((3, F, U)), _full_spec((1, U)),
            _full_spec((U, 1)), _full_spec((1, 1)),
        ],
        out_specs=[_row_spec(U), _row_spec(1)],
        out_shape=[
            jax.ShapeDtypeStruct((8 * N, U), jnp.float32),
            jax.ShapeDtypeStruct((8 * N, 1), jnp.float32),
        ],
    )(x0, x1, x2, u, hx, w, b, wp, bp)


def _spmm_sc_body(x_hbm, xneg_hbm, zero_hbm, colg_hbm, row_hbm, val_hbm,
                  y1_hbm, y2_hbm,
                  acc, colbs, rowbs, valbs, rowsbs,
                  isems, gsems, ssems):
    ci = lax.axis_index("c")
    si = lax.axis_index("s")
    t0 = si * SROWS
    e0 = si * EPT
    nf = F // 16

    def edge_pass(src_hbm, bidx, dbl):
        cbase = bidx * E + e0

        def in_args(w, s):
            return (
                (colg_hbm.at[pl.ds(cbase + w * EW, EW)], colbs[s], isems[s]),
                (row_hbm.at[pl.ds(e0 + w * EW, EW)], rowbs[s], isems[s]),
                (val_hbm.at[pl.ds((e0 + w * EW) * 16, EW * 16)], valbs[s], isems[s]),
            )

        def issue_inputs(w, s):
            for a in in_args(w, s):
                pltpu.async_copy(*a)

        def wait_inputs(w, s):
            for a in in_args(w, s):
                pltpu.make_async_copy(*a).wait()

        def gather_start(s):
            pltpu.async_copy(src_hbm.at[colbs[s]], rowsbs[s], gsems[s])

        def gather_wait(s):
            pltpu.make_async_copy(src_hbm.at[colbs[s]], rowsbs[s], gsems[s]).wait()

        def scatter_start(s):
            pltpu.async_copy(rowsbs[s], acc.at[rowbs[s]], ssems[s], add=True)

        def scatter_wait(s):
            pltpu.make_async_copy(rowsbs[s], acc.at[rowbs[s]], ssems[s]).wait()

        def scale(s):
            rb, vb = rowsbs[s], valbs[s]

            def scl(k2, c2):
                for uu in range(4):
                    k = k2 * 4 + uu
                    v = vb[pl.ds(k * 16, 16)]
                    if dbl:
                        v = v + v
                    for jj in range(NFS):
                        sl = pl.ds(jj * 16, 16)
                        rb[k, sl] = rb[k, sl] * v
                return c2

            lax.fori_loop(0, EW // 4, scl, 0)

        issue_inputs(0, 0)
        issue_inputs(1, 1)
        issue_inputs(2, 2)
        wait_inputs(0, 0)
        gather_start(0)
        wait_inputs(1, 1)
        gather_start(1)

        def quint(q, carry):
            w0 = q * NSLOT
            for s_ in range(NSLOT):
                w = w0 + s_
                sp2 = (s_ + 2) % NSLOT
                sp3 = (s_ + 3) % NSLOT

                @pl.when(w >= 2)
                def _():
                    scatter_wait(sp3)

                @pl.when(w + 3 < NWIN)
                def _():
                    issue_inputs(w + 3, sp3)

                @pl.when(w + 2 < NWIN)
                def _():
                    wait_inputs(w + 2, sp2)
                    gather_start(sp2)

                gather_wait(s_)
                scale(s_)
                scatter_start(s_)
            return carry

        lax.fori_loop(0, NWIN // NSLOT, quint, 0)
        scatter_wait((NWIN - 2) % NSLOT)
        scatter_wait((NWIN - 1) % NSLOT)

    def zero_stripe(off, size):
        pltpu.sync_copy(zero_hbm.at[pl.ds(off, size)], acc.at[pl.ds(off, size)])

    def negx_stripe(base, off, size):
        pltpu.sync_copy(xneg_hbm.at[pl.ds(base + off, size)],
                        acc.at[pl.ds(off, size)])

    def writeback_stripe(y_hbm, base, off, size):
        pltpu.sync_copy(acc.at[pl.ds(off, size)], y_hbm.at[pl.ds(base + off, size)])

    def all_stripes(fn, *args):
        fn(*args, t0, SROWS)

        @pl.when(si == 0)
        def _():
            fn(*args, NS * SROWS, TAIL)

    def chunk(j, carry):
        bidx = ci * CPS + j
        base = bidx * N

        all_stripes(zero_stripe)
        plsc.subcore_barrier()
        edge_pass(x_hbm, bidx, False)
        plsc.subcore_barrier()
        all_stripes(writeback_stripe, y1_hbm, base)
        plsc.subcore_barrier()

        all_stripes(negx_stripe, base)
        plsc.subcore_barrier()
        edge_pass(y1_hbm, bidx, True)
        plsc.subcore_barrier()
        all_stripes(writeback_stripe, y2_hbm, base)
        plsc.subcore_barrier()
        return carry

    lax.fori_loop(0, CPS, chunk, 0)


_spmm_sc_call = pl.kernel(
    _spmm_sc_body,
    out_type=[
        jax.ShapeDtypeStruct((B * N, F), jnp.float32),
        jax.ShapeDtypeStruct((B * N, F), jnp.float32),
    ],
    mesh=plsc.VectorSubcoreMesh(core_axis_name="c", subcore_axis_name="s"),
    compiler_params=pltpu.CompilerParams(needs_layout_passes=False),
    scratch_types=[
        pltpu.VMEM_SHARED((N, F), jnp.float32),
        [pltpu.VMEM((EW,), jnp.int32) for _ in range(NSLOT)],
        [pltpu.VMEM((EW,), jnp.int32) for _ in range(NSLOT)],
        [pltpu.VMEM((EW * 16,), jnp.float32) for _ in range(NSLOT)],
        [pltpu.VMEM((EW, F), jnp.float32) for _ in range(NSLOT)],
        [pltpu.SemaphoreType.DMA for _ in range(NSLOT)],
        [pltpu.SemaphoreType.DMA for _ in range(NSLOT)],
        [pltpu.SemaphoreType.DMA for _ in range(NSLOT)],
    ],
)


def _spmm_pair(row, colg8, val, zeros_n, x):
    xf = x.reshape(B * N, F)
    y1, y2 = _spmm_sc_call(xf, -xf, zeros_n, colg8, row, val)
    return y1.reshape(B, N, F), y2.reshape(B, N, F)


def _prep_w(W, out_dim):
    Wk = W.reshape(U + 1, 3, out_dim).transpose(1, 0, 2)
    return jnp.pad(Wk, ((0, 0), (0, F - 1 - U), (0, 0)))


def kernel(inputs, hidden_state, edge_row, edge_col, lap_val, W_gate, b_gate,
           W_cand, b_cand, W_proj, b_proj):
    inp = inputs.reshape(B, N, 1)
    hx = hidden_state[0].reshape(B, N, U)
    x0 = jnp.concatenate([inp, hx, jnp.zeros((B, N, F - 1 - U), jnp.float32)], axis=2)

    wg = _prep_w(W_gate, GATE)
    wc = _prep_w(W_cand, U)

    colg8 = (edge_col[None, :]
             + (jnp.arange(B, dtype=jnp.int32) * N)[:, None]).reshape(-1)
    valx = jnp.broadcast_to(lap_val[:, None], (E, 16)).reshape(E * 16)
    zeros_n = jnp.zeros((N, F), jnp.float32)

    x1, x2 = _spmm_pair(edge_row, colg8, valx, zeros_n, x0)
    xc, u = _gate_call(
        x0.reshape(8 * N, F), x1.reshape(8 * N, F), x2.reshape(8 * N, F),
        hx.reshape(8 * N, U), wg, b_gate.reshape(1, GATE))

    xcb = xc.reshape(B, N, F)
    xc1, xc2 = _spmm_pair(edge_row, colg8, valx, zeros_n, xcb)
    h, p = _cand_call(
        xc, xc1.reshape(8 * N, F), xc2.reshape(8 * N, F), u,
        hx.reshape(8 * N, U), wc, b_cand.reshape(1, U),
        W_proj, b_proj.reshape(1, 1))

    out = p.reshape(B, N)
    hidden = h.reshape(1, B, N * U)
    return (out, hidden)


# speedup vs baseline: 3.3050x; 1.0001x over previous
import jax
import jax.numpy as jnp
from jax import lax
from jax.experimental import pallas as pl
from jax.experimental.pallas import tpu as pltpu
from jax.experimental.pallas import tpu_sc as plsc

N = 10000
E = 320000
B = 8
U = 64
F = 128
GATE = 2 * U

BN = 2000

NC = 2
NS = 16
SROWS = 624
TAIL = N - NS * SROWS
EPT = E // NS
EW = 40
NWIN = EPT // EW
NSLOT = 5
NFS = 5
CPS = B // NC


def _gate_body(x0_ref, x1_ref, x2_ref, hx_ref, w_ref, b_ref, xc_ref, u_ref):
    acc = (
        jnp.dot(x0_ref[...], w_ref[0], preferred_element_type=jnp.float32)
        + jnp.dot(x1_ref[...], w_ref[1], preferred_element_type=jnp.float32)
        + jnp.dot(x2_ref[...], w_ref[2], preferred_element_type=jnp.float32)
        + b_ref[...]
    )
    v = jax.nn.sigmoid(acc)
    r = v[:, :U]
    u = v[:, U:]
    u_ref[...] = u
    rh = r * hx_ref[...]
    xc_ref[...] = jnp.concatenate(
        [x0_ref[:, 0:1], rh, jnp.zeros((BN, F - 1 - U), jnp.float32)], axis=1
    )


def _cand_body(x0_ref, x1_ref, x2_ref, u_ref, hx_ref, w_ref, b_ref, wp_ref,
               bp_ref, h_ref, p_ref):
    acc = (
        jnp.dot(x0_ref[...], w_ref[0], preferred_element_type=jnp.float32)
        + jnp.dot(x1_ref[...], w_ref[1], preferred_element_type=jnp.float32)
        + jnp.dot(x2_ref[...], w_ref[2], preferred_element_type=jnp.float32)
        + b_ref[...]
    )
    c = jnp.tanh(acc)
    u = u_ref[...]
    h = u * hx_ref[...] + (1.0 - u) * c
    h_ref[...] = h
    p_ref[...] = jnp.dot(h, wp_ref[...], preferred_element_type=jnp.float32) + bp_ref[...]


def _row_spec(width):
    return pl.BlockSpec((BN, width), lambda i: (i, 0))


def _full_spec(shape):
    return pl.BlockSpec(shape, lambda i: tuple(0 for _ in shape))


def _gate_call(x0, x1, x2, hx, w, b):
    grid = (8 * N // BN,)
    return pl.pallas_call(
        _gate_body,
        grid=grid,
        in_specs=[
            _row_spec(F), _row_spec(F), _row_spec(F), _row_spec(U),
            _full_spec((3, F, GATE)), _full_spec((1, GATE)),
        ],
        out_specs=[_row_spec(F), _row_spec(U)],
        out_shape=[
            jax.ShapeDtypeStruct((8 * N, F), jnp.float32),
            jax.ShapeDtypeStruct((8 * N, U), jnp.float32),
        ],
    )(x0, x1, x2, hx, w, b)


def _cand_call(x0, x1, x2, u, hx, w, b, wp, bp):
    grid = (8 * N // BN,)
    return pl.pallas_call(
        _cand_body,
        grid=grid,
        in_specs=[
            _row_spec(F), _row_spec(F), _row_spec(F), _row_spec(U), _row_spec(U),
            _full_spec((3, F, U)), _full_spec((1, U)),
            _full_spec((U, 1)), _full_spec((1, 1)),
        ],
        out_specs=[_row_spec(U), _row_spec(1)],
        out_shape=[
            jax.ShapeDtypeStruct((8 * N, U), jnp.float32),
            jax.ShapeDtypeStruct((8 * N, 1), jnp.float32),
        ],
    )(x0, x1, x2, u, hx, w, b, wp, bp)


def _spmm_sc_body(x_hbm, xneg_hbm, zero_hbm, colg_hbm, row_hbm, val_hbm,
                  y1_hbm, y2_hbm,
                  acc, colbs, rowbs, valbs, rowsbs,
                  isems, gsems, ssems):
    ci = lax.axis_index("c")
    si = lax.axis_index("s")
    t0 = si * SROWS
    e0 = si * EPT
    nf = F // 16

    def edge_pass(src_hbm, bidx, dbl):
        cbase = bidx * E + e0

        def in_args(w, s):
            return (
                (colg_hbm.at[pl.ds(cbase + w * EW, EW)], colbs[s], isems[s]),
                (row_hbm.at[pl.ds(e0 + w * EW, EW)], rowbs[s], isems[s]),
                (val_hbm.at[pl.ds((e0 + w * EW) * 16, EW * 16)], valbs[s], isems[s]),
            )

        def issue_inputs(w, s):
            for a in in_args(w, s):
                pltpu.async_copy(*a)

        def wait_inputs(w, s):
            for a in in_args(w, s):
                pltpu.make_async_copy(*a).wait()

        def gather_start(s):
            pltpu.async_copy(src_hbm.at[colbs[s]], rowsbs[s], gsems[s])

        def gather_wait(s):
            pltpu.make_async_copy(src_hbm.at[colbs[s]], rowsbs[s], gsems[s]).wait()

        def scatter_start(s):
            pltpu.async_copy(rowsbs[s], acc.at[rowbs[s]], ssems[s], add=True)

        def scatter_wait(s):
            pltpu.make_async_copy(rowsbs[s], acc.at[rowbs[s]], ssems[s]).wait()

        def scale(s):
            rb, vb = rowsbs[s], valbs[s]

            def scl(k2, c2):
                for uu in range(4):
                    k = k2 * 4 + uu
                    v = vb[pl.ds(k * 16, 16)]
                    if dbl:
                        v = v + v
                    for jj in range(NFS):
                        sl = pl.ds(jj * 16, 16)
                        rb[k, sl] = rb[k, sl] * v
                return c2

            lax.fori_loop(0, EW // 4, scl, 0)

        issue_inputs(0, 0)
        issue_inputs(1, 1)
        issue_inputs(2, 2)
        wait_inputs(0, 0)
        gather_start(0)
        wait_inputs(1, 1)
        gather_start(1)

        def quint(q, carry):
            w0 = q * NSLOT
            for s_ in range(NSLOT):
                w = w0 + s_
                sp2 = (s_ + 2) % NSLOT
                sp3 = (s_ + 3) % NSLOT

                @pl.when(w >= 2)
                def _():
                    scatter_wait(sp3)

                @pl.when(w + 3 < NWIN)
                def _():
                    issue_inputs(w + 3, sp3)

                @pl.when(w + 2 < NWIN)
                def _():
                    wait_inputs(w + 2, sp2)
                    gather_start(sp2)

                gather_wait(s_)
                scale(s_)
                scatter_start(s_)
            return carry

        lax.fori_loop(0, NWIN // NSLOT, quint, 0)
        scatter_wait((NWIN - 2) % NSLOT)
        scatter_wait((NWIN - 1) % NSLOT)

    def zero_stripe(off, size):
        pltpu.sync_copy(zero_hbm.at[pl.ds(off, size)], acc.at[pl.ds(off, size)])

    def negx_stripe(base, off, size):
        pltpu.sync_copy(xneg_hbm.at[pl.ds(base + off, size)],
                        acc.at[pl.ds(off, size)])

    def writeback_stripe(y_hbm, base, off, size):
        pltpu.sync_copy(acc.at[pl.ds(off, size)], y_hbm.at[pl.ds(base + off, size)])

    def all_stripes(fn, *args):
        fn(*args, t0, SROWS)

        @pl.when(si == 0)
        def _():
            fn(*args, NS * SROWS, TAIL)

    def chunk(j, carry):
        bidx = ci * CPS + j
        base = bidx * N

        all_stripes(zero_stripe)
        plsc.subcore_barrier()
        edge_pass(x_hbm, bidx, False)
        plsc.subcore_barrier()
        all_stripes(writeback_stripe, y1_hbm, base)
        plsc.subcore_barrier()

        all_stripes(negx_stripe, base)
        plsc.subcore_barrier()
        edge_pass(y1_hbm, bidx, True)
        plsc.subcore_barrier()
        all_stripes(writeback_stripe, y2_hbm, base)
        plsc.subcore_barrier()
        return carry

    lax.fori_loop(0, CPS, chunk, 0)


_spmm_sc_call = pl.kernel(
    _spmm_sc_body,
    out_type=[
        jax.ShapeDtypeStruct((B * N, F), jnp.float32),
        jax.ShapeDtypeStruct((B * N, F), jnp.float32),
    ],
    mesh=plsc.VectorSubcoreMesh(core_axis_name="c", subcore_axis_name="s"),
    compiler_params=pltpu.CompilerParams(needs_layout_passes=False),
    scratch_types=[
        pltpu.VMEM_SHARED((N, F), jnp.float32),
        [pltpu.VMEM((EW,), jnp.int32) for _ in range(NSLOT)],
        [pltpu.VMEM((EW,), jnp.int32) for _ in range(NSLOT)],
        [pltpu.VMEM((EW * 16,), jnp.float32) for _ in range(NSLOT)],
        [pltpu.VMEM((EW, F), jnp.float32) for _ in range(NSLOT)],
        [pltpu.SemaphoreType.DMA for _ in range(NSLOT)],
        [pltpu.SemaphoreType.DMA for _ in range(NSLOT)],
        [pltpu.SemaphoreType.DMA for _ in range(NSLOT)],
    ],
)


def _spmm_pair(row, colg8, val, zeros_n, x):
    xf = x.reshape(B * N, F)
    y1, y2 = _spmm_sc_call(xf, -xf, zeros_n, colg8, row, val)
    return y1.reshape(B, N, F), y2.reshape(B, N, F)


def _prep_w(W, out_dim):
    Wk = W.reshape(U + 1, 3, out_dim).transpose(1, 0, 2)
    return jnp.pad(Wk, ((0, 0), (0, F - 1 - U), (0, 0)))


def kernel(inputs, hidden_state, edge_row, edge_col, lap_val, W_gate, b_gate,
           W_cand, b_cand, W_proj, b_proj):
    inp = inputs.reshape(B, N, 1)
    hx = hidden_state[0].reshape(B, N, U)
    x0 = jnp.concatenate([inp, hx, jnp.zeros((B, N, F - 1 - U), jnp.float32)], axis=2)

    wg = _prep_w(W_gate, GATE)
    wc = _prep_w(W_cand, U)

    colg8 = (edge_col[None, :]
             + (jnp.arange(B, dtype=jnp.int32) * N)[:, None]).reshape(-1)
    valx = jnp.broadcast_to(lap_val[:, None], (E, 16)).reshape(E * 16)
    zeros_n = jnp.zeros((N, F), jnp.float32)

    x1, x2 = _spmm_pair(edge_row, colg8, valx, zeros_n, x0)
    xc, u = _gate_call(
        x0.reshape(8 * N, F), x1.reshape(8 * N, F), x2.reshape(8 * N, F),
        hx.reshape(8 * N, U), wg, b_gate.reshape(1, GATE))

    xcb = xc.reshape(B, N, F)
    xc1, xc2 = _spmm_pair(edge_row, colg8, valx, zeros_n, xcb)
    h, p = _cand_call(
        xc, xc1.reshape(8 * N, F), xc2.reshape(8 * N, F), u,
        hx.reshape(8 * N, U), wc, b_cand.reshape(1, U),
        W_proj, b_proj.reshape(1, 1))

    out = p.reshape(B, N)
    hidden = h.reshape(1, B, N * U)
    return (out, hidden)
